# Initial kernel scaffold; baseline (speedup 1.0000x reference)
import jax
import jax.numpy as jnp
from jax.experimental import pallas as pl


def kernel(x, edge_index, W1_l, b1, W1_r, W2_l, b2, W2_r, Wc, bc):
    raise NotImplementedError("write your pallas kernel here")


# speedup vs baseline: 4.3334x; 4.3334x over previous
import functools

import jax
import jax.numpy as jnp
from jax import lax
from jax.experimental import pallas as pl
from jax.experimental.pallas import tpu as pltpu
from jax.experimental.pallas import tpu_sc as plsc

_NC, _NS = 2, 16
_NW = _NC * _NS
_LANES = 16


def _sc_mesh():
    return plsc.VectorSubcoreMesh(
        core_axis_name="c", subcore_axis_name="s",
        num_cores=_NC, num_subcores=_NS)


def _make_sc_agg(n_pad, d, nb, nbc, k, with_deg):
    rpt = n_pad // _NS

    out_type = [jax.ShapeDtypeStruct((_NC, n_pad, d), jnp.float32)]
    scratch = [
        pltpu.VMEM((nbc, k), jnp.int32),
        pltpu.VMEM((nbc, k), jnp.int32),
        pltpu.VMEM((k, d), jnp.float32),
        pltpu.VMEM_SHARED((n_pad, d), jnp.float32),
    ]
    if with_deg:
        out_type.append(jax.ShapeDtypeStruct((_NW, 1, n_pad), jnp.float32))
        scratch.append(pltpu.VMEM((n_pad,), jnp.float32))

    def body(*refs):
        if with_deg:
            (src_hbm, row_hbm, col_hbm, zrow_hbm, zdeg_hbm,
             agg_hbm, deg_hbm, row_v, col_v, gbuf, acc_sh, deg_v) = refs
        else:
            (src_hbm, row_hbm, col_hbm, zrow_hbm,
             agg_hbm, row_v, col_v, gbuf, acc_sh) = refs
        c = lax.axis_index("c")
        s = lax.axis_index("s")
        wid = c * _NS + s

        pltpu.sync_copy(zrow_hbm, acc_sh.at[pl.ds(s * rpt, rpt)])
        if with_deg:
            pltpu.sync_copy(zdeg_hbm.at[0], deg_v)
        plsc.subcore_barrier()

        ones16 = jnp.ones((_LANES,), jnp.float32)

        def chunk(ci, carry):
            pltpu.sync_copy(row_hbm.at[wid, pl.ds(ci * nbc, nbc)], row_v)
            pltpu.sync_copy(col_hbm.at[wid, pl.ds(ci * nbc, nbc)], col_v)

            def step(b, carry2):
                pltpu.sync_copy(src_hbm.at[row_v.at[b]], gbuf)
                pltpu.sync_copy(gbuf, acc_sh.at[col_v.at[b]], add=True)
                if with_deg:
                    for j in range(k // _LANES):
                        cj = col_v[b, pl.ds(j * _LANES, _LANES)]
                        plsc.addupdate_scatter(deg_v, [cj], ones16)
                return carry2

            return lax.fori_loop(0, nbc, step, carry)

        lax.fori_loop(0, nb // nbc, chunk, 0)

        plsc.subcore_barrier()
        pltpu.sync_copy(acc_sh.at[pl.ds(s * rpt, rpt)],
                        agg_hbm.at[c, pl.ds(s * rpt, rpt)])
        if with_deg:
            pltpu.sync_copy(deg_v, deg_hbm.at[wid, 0])

    return pl.kernel(body, out_type=out_type, mesh=_sc_mesh(),
                     scratch_types=scratch,
                     compiler_params=pltpu.CompilerParams(
                         needs_layout_passes=False))


def _make_sc_edge(n_pad, e, ep):
    ni = ep // _LANES
    out_type = [jax.ShapeDtypeStruct((2 * e,), jnp.float32)]
    scratch = [
        pltpu.VMEM((4 * n_pad,), jnp.float32),
        pltpu.VMEM((ep,), jnp.int32),
        pltpu.VMEM((ep,), jnp.int32),
        pltpu.VMEM((2 * ep,), jnp.float32),
    ]

    def body(pq_hbm, row_hbm, col_hbm, out_hbm, pq_v, row_v, col_v, out_v):
        c = lax.axis_index("c")
        s = lax.axis_index("s")
        wid = c * _NS + s
        base = wid * ep

        pltpu.sync_copy(pq_hbm, pq_v)
        pltpu.sync_copy(row_hbm.at[pl.ds(base, ep)], row_v)
        pltpu.sync_copy(col_hbm.at[pl.ds(base, ep)], col_v)

        iota2 = lax.iota(jnp.int32, _LANES) * 2
        c1 = jnp.ones((_LANES,), jnp.int32)

        def step(i, carry):
            r = row_v[pl.ds(i * _LANES, _LANES)] * 4
            cc = col_v[pl.ds(i * _LANES, _LANES)] * 4 + 2
            o0 = plsc.load_gather(pq_v, [r]) + plsc.load_gather(pq_v, [cc])
            o1 = (plsc.load_gather(pq_v, [r + c1])
                  + plsc.load_gather(pq_v, [cc + c1]))
            pos = i * (2 * _LANES) + iota2
            plsc.store_scatter(out_v, [pos], o0)
            plsc.store_scatter(out_v, [pos + c1], o1)
            return carry

        lax.fori_loop(0, ni, step, 0)
        pltpu.sync_copy(out_v, out_hbm.at[pl.ds(2 * base, 2 * ep)])

    return pl.kernel(body, out_type=out_type, mesh=_sc_mesh(),
                     scratch_types=scratch,
                     compiler_params=pltpu.CompilerParams(
                         needs_layout_passes=False))


def _tc_blk(n_pad):
    best = 8
    for cand in range(8, 2049, 8):
        if n_pad % cand == 0:
            best = cand
    return best


def _tc_layer1(agg, deg_t, x, w_l, w_r, b):
    n_pad, d = x.shape
    blk = _tc_blk(n_pad)
    rs = lambda i: (i, 0)
    full = lambda i: (0, 0)

    def body(a0_ref, a1_ref, degt_ref, x_ref, wl_ref, wr_ref, b_ref,
             h_ref, inv_ref):
        deg = jnp.sum(degt_ref[...], axis=1, keepdims=True)
        inv = 1.0 / jnp.maximum(deg, 1.0)
        mean = (a0_ref[0] + a1_ref[0]) * inv
        h = (jnp.dot(mean, wl_ref[...], preferred_element_type=jnp.float32)
             + jnp.dot(x_ref[...], wr_ref[...], preferred_element_type=jnp.float32)
             + b_ref[...])
        h_ref[...] = jnp.maximum(h, 0.0)
        inv_ref[...] = inv

    return pl.pallas_call(
        body,
        grid=(n_pad // blk,),
        in_specs=[
            pl.BlockSpec((1, blk, d), lambda i: (0, i, 0)),
            pl.BlockSpec((1, blk, d), lambda i: (1, i, 0)),
            pl.BlockSpec((blk, _NW), rs),
            pl.BlockSpec((blk, d), rs),
            pl.BlockSpec((d, d), full),
            pl.BlockSpec((d, d), full),
            pl.BlockSpec((1, d), full),
        ],
        out_specs=[pl.BlockSpec((blk, d), rs), pl.BlockSpec((blk, 1), rs)],
        out_shape=[jax.ShapeDtypeStruct((n_pad, d), jnp.float32),
                   jax.ShapeDtypeStruct((n_pad, 1), jnp.float32)],
    )(agg, agg, deg_t, x, w_l, w_r, b)


def _tc_layer2(agg, inv, h1, w_l, w_r, b, wc2, bc4):
    n_pad, d = h1.shape
    blk = _tc_blk(n_pad)
    rs = lambda i: (i, 0)
    full = lambda i: (0, 0)

    def body(a0_ref, a1_ref, inv_ref, h1_ref, wl_ref, wr_ref, b_ref,
             wc_ref, bc_ref, pq_ref):
        mean = (a0_ref[0] + a1_ref[0]) * inv_ref[...]
        h2 = (jnp.dot(mean, wl_ref[...], preferred_element_type=jnp.float32)
              + jnp.dot(h1_ref[...], wr_ref[...], preferred_element_type=jnp.float32)
              + b_ref[...])
        pq_ref[...] = (jnp.dot(h2, wc_ref[...],
                               preferred_element_type=jnp.float32)
                       + bc_ref[...])

    return pl.pallas_call(
        body,
        grid=(n_pad // blk,),
        in_specs=[
            pl.BlockSpec((1, blk, d), lambda i: (0, i, 0)),
            pl.BlockSpec((1, blk, d), lambda i: (1, i, 0)),
            pl.BlockSpec((blk, 1), rs),
            pl.BlockSpec((blk, d), rs),
            pl.BlockSpec((d, d), full),
            pl.BlockSpec((d, d), full),
            pl.BlockSpec((1, d), full),
            pl.BlockSpec((d, 4), full),
            pl.BlockSpec((1, 4), full),
        ],
        out_specs=[pl.BlockSpec((blk, 4), rs)],
        out_shape=[jax.ShapeDtypeStruct((n_pad, 4), jnp.float32)],
    )(agg, agg, inv, h1, w_l, w_r, b, wc2, bc4)


def kernel(x, edge_index, W1_l, b1, W1_r, W2_l, b2, W2_r, Wc, bc):
    n, d = x.shape
    e = edge_index.shape[1]
    ep = e // _NW
    k = 128
    nbc = 16
    nb = -(-ep // k)
    nb = -(-nb // nbc) * nbc
    pad = nb * k - ep
    assert ep % _LANES == 0 and (ep * 4) % 8 == 0

    n_pad = -(-(n + 1) // 128) * 128
    row = edge_index[0].astype(jnp.int32)
    col = edge_index[1].astype(jnp.int32)
    row_p = jnp.pad(row.reshape(_NW, ep), ((0, 0), (0, pad))).reshape(
        _NW, nb, k)
    dummy = (n + jnp.arange(_NW, dtype=jnp.int32) % (n_pad - n))[:, None]
    col_p = jnp.concatenate(
        [col.reshape(_NW, ep),
         jnp.broadcast_to(dummy, (_NW, pad))], axis=1).reshape(_NW, nb, k)
    xp = jnp.pad(x, ((0, n_pad - n), (0, 0)))
    zrow = jnp.zeros((n_pad // _NS, d), jnp.float32)
    zdeg = jnp.zeros((1, n_pad), jnp.float32)

    agg1, degp = _make_sc_agg(n_pad, d, nb, nbc, k, True)(
        xp, row_p, col_p, zrow, zdeg)
    deg_t = degp.reshape(_NW, n_pad).T
    h1, inv = _tc_layer1(agg1, deg_t, xp, W1_l, W1_r, b1.reshape(1, d))

    (agg2,) = _make_sc_agg(n_pad, d, nb, nbc, k, False)(
        h1, row_p, col_p, zrow)
    wc2 = jnp.concatenate([Wc[:d], Wc[d:]], axis=1)
    bc4 = jnp.concatenate([bc, jnp.zeros((2,), jnp.float32)]).reshape(1, 4)
    (pq,) = _tc_layer2(agg2, inv, h1, W2_l, W2_r, b2.reshape(1, d), wc2, bc4)

    (out,) = _make_sc_edge(n_pad, e, ep)(pq.reshape(4 * n_pad), row, col)
    return out.reshape(e, 2)


# speedup vs baseline: 4.5478x; 1.0495x over previous
import functools

import jax
import jax.numpy as jnp
from jax import lax
from jax.experimental import pallas as pl
from jax.experimental.pallas import tpu as pltpu
from jax.experimental.pallas import tpu_sc as plsc

_NC, _NS = 2, 16
_NW = _NC * _NS
_LANES = 16


def _sc_mesh():
    return plsc.VectorSubcoreMesh(
        core_axis_name="c", subcore_axis_name="s",
        num_cores=_NC, num_subcores=_NS)


def _make_sc_agg(n_pad, d, nb, nbc, k, with_deg):
    rpt = n_pad // _NS

    out_type = [jax.ShapeDtypeStruct((_NC, n_pad, d), jnp.float32)]
    scratch = [
        pltpu.VMEM((nbc, k), jnp.int32),
        pltpu.VMEM((nbc, k), jnp.int32),
        pltpu.VMEM((k, d), jnp.float32),
        pltpu.VMEM((k, d), jnp.float32),
        pltpu.SemaphoreType.DMA,
        pltpu.SemaphoreType.DMA,
        pltpu.VMEM_SHARED((n_pad, d), jnp.float32),
    ]
    if with_deg:
        out_type.append(jax.ShapeDtypeStruct((_NW, 1, n_pad), jnp.float32))
        scratch.append(pltpu.VMEM((n_pad,), jnp.float32))

    def body(*refs):
        deg_v = None
        if with_deg:
            (src_hbm, row_hbm, col_hbm, zrow_hbm, zdeg_hbm,
             agg_hbm, deg_hbm, row_v, col_v, g0, g1, sg0, sg1,
             acc_sh, deg_v) = refs
        else:
            (src_hbm, row_hbm, col_hbm, zrow_hbm,
             agg_hbm, row_v, col_v, g0, g1, sg0, sg1, acc_sh) = refs
        gbufs, sems = (g0, g1), (sg0, sg1)
        c = lax.axis_index("c")
        s = lax.axis_index("s")
        wid = c * _NS + s

        pltpu.sync_copy(zrow_hbm, acc_sh.at[pl.ds(s * rpt, rpt)])
        if with_deg:
            pltpu.sync_copy(zdeg_hbm.at[0], deg_v)
        plsc.subcore_barrier()

        ones16 = jnp.ones((_LANES,), jnp.float32)
        nbuf = 2

        def chunk(ci, carry):
            pltpu.sync_copy(row_hbm.at[wid, pl.ds(ci * nbc, nbc)], row_v)
            pltpu.sync_copy(col_hbm.at[wid, pl.ds(ci * nbc, nbc)], col_v)
            cps = [
                pltpu.async_copy(src_hbm.at[row_v.at[j]], gbufs[j], sems[j])
                for j in range(nbuf)
            ]
            for j in range(nbc):
                jj = j % nbuf
                if with_deg:
                    for t in range(k // _LANES):
                        cj = col_v[j, pl.ds(t * _LANES, _LANES)]
                        plsc.addupdate_scatter(deg_v, [cj], ones16)
                cps[jj].wait()
                pltpu.sync_copy(gbufs[jj], acc_sh.at[col_v.at[j]], add=True)
                if j + nbuf < nbc:
                    cps[jj] = pltpu.async_copy(
                        src_hbm.at[row_v.at[j + nbuf]], gbufs[jj], sems[jj])
            return carry

        lax.fori_loop(0, nb // nbc, chunk, 0)

        plsc.subcore_barrier()
        pltpu.sync_copy(acc_sh.at[pl.ds(s * rpt, rpt)],
                        agg_hbm.at[c, pl.ds(s * rpt, rpt)])
        if with_deg:
            pltpu.sync_copy(deg_v, deg_hbm.at[wid, 0])

    return pl.kernel(body, out_type=out_type, mesh=_sc_mesh(),
                     scratch_types=scratch,
                     compiler_params=pltpu.CompilerParams(
                         needs_layout_passes=False))


def _make_sc_edge(n_pad, e, ep):
    ni = ep // _LANES
    out_type = [jax.ShapeDtypeStruct((2 * e,), jnp.float32)]
    scratch = [
        pltpu.VMEM((4 * n_pad,), jnp.float32),
        pltpu.VMEM((ep,), jnp.int32),
        pltpu.VMEM((ep,), jnp.int32),
        pltpu.VMEM((2 * ep,), jnp.float32),
    ]

    def body(pq_hbm, row_hbm, col_hbm, out_hbm, pq_v, row_v, col_v, out_v):
        c = lax.axis_index("c")
        s = lax.axis_index("s")
        wid = c * _NS + s
        base = wid * ep

        pltpu.sync_copy(pq_hbm, pq_v)
        pltpu.sync_copy(row_hbm.at[pl.ds(base, ep)], row_v)
        pltpu.sync_copy(col_hbm.at[pl.ds(base, ep)], col_v)

        iota2 = lax.iota(jnp.int32, _LANES) * 2
        c1 = jnp.ones((_LANES,), jnp.int32)

        def step(i, carry):
            r = row_v[pl.ds(i * _LANES, _LANES)] * 4
            cc = col_v[pl.ds(i * _LANES, _LANES)] * 4 + 2
            o0 = plsc.load_gather(pq_v, [r]) + plsc.load_gather(pq_v, [cc])
            o1 = (plsc.load_gather(pq_v, [r + c1])
                  + plsc.load_gather(pq_v, [cc + c1]))
            pos = i * (2 * _LANES) + iota2
            plsc.store_scatter(out_v, [pos], o0)
            plsc.store_scatter(out_v, [pos + c1], o1)
            return carry

        lax.fori_loop(0, ni, step, 0)
        pltpu.sync_copy(out_v, out_hbm.at[pl.ds(2 * base, 2 * ep)])

    return pl.kernel(body, out_type=out_type, mesh=_sc_mesh(),
                     scratch_types=scratch,
                     compiler_params=pltpu.CompilerParams(
                         needs_layout_passes=False))


def _tc_blk(n_pad):
    best = 8
    for cand in range(8, 2049, 8):
        if n_pad % cand == 0:
            best = cand
    return best


def _tc_layer1(agg, deg_t, x, w_l, w_r, b):
    n_pad, d = x.shape
    blk = _tc_blk(n_pad)
    rs = lambda i: (i, 0)
    full = lambda i: (0, 0)

    def body(a0_ref, a1_ref, degt_ref, x_ref, wl_ref, wr_ref, b_ref,
             h_ref, inv_ref):
        deg = jnp.sum(degt_ref[...], axis=1, keepdims=True)
        inv = 1.0 / jnp.maximum(deg, 1.0)
        mean = (a0_ref[0] + a1_ref[0]) * inv
        h = (jnp.dot(mean, wl_ref[...], preferred_element_type=jnp.float32)
             + jnp.dot(x_ref[...], wr_ref[...], preferred_element_type=jnp.float32)
             + b_ref[...])
        h_ref[...] = jnp.maximum(h, 0.0)
        inv_ref[...] = inv

    return pl.pallas_call(
        body,
        grid=(n_pad // blk,),
        in_specs=[
            pl.BlockSpec((1, blk, d), lambda i: (0, i, 0)),
            pl.BlockSpec((1, blk, d), lambda i: (1, i, 0)),
            pl.BlockSpec((blk, _NW), rs),
            pl.BlockSpec((blk, d), rs),
            pl.BlockSpec((d, d), full),
            pl.BlockSpec((d, d), full),
            pl.BlockSpec((1, d), full),
        ],
        out_specs=[pl.BlockSpec((blk, d), rs), pl.BlockSpec((blk, 1), rs)],
        out_shape=[jax.ShapeDtypeStruct((n_pad, d), jnp.float32),
                   jax.ShapeDtypeStruct((n_pad, 1), jnp.float32)],
    )(agg, agg, deg_t, x, w_l, w_r, b)


def _tc_layer2(agg, inv, h1, w_l, w_r, b, wc2, bc4):
    n_pad, d = h1.shape
    blk = _tc_blk(n_pad)
    rs = lambda i: (i, 0)
    full = lambda i: (0, 0)

    def body(a0_ref, a1_ref, inv_ref, h1_ref, wl_ref, wr_ref, b_ref,
             wc_ref, bc_ref, pq_ref):
        mean = (a0_ref[0] + a1_ref[0]) * inv_ref[...]
        h2 = (jnp.dot(mean, wl_ref[...], preferred_element_type=jnp.float32)
              + jnp.dot(h1_ref[...], wr_ref[...], preferred_element_type=jnp.float32)
              + b_ref[...])
        pq_ref[...] = (jnp.dot(h2, wc_ref[...],
                               preferred_element_type=jnp.float32)
                       + bc_ref[...])

    return pl.pallas_call(
        body,
        grid=(n_pad // blk,),
        in_specs=[
            pl.BlockSpec((1, blk, d), lambda i: (0, i, 0)),
            pl.BlockSpec((1, blk, d), lambda i: (1, i, 0)),
            pl.BlockSpec((blk, 1), rs),
            pl.BlockSpec((blk, d), rs),
            pl.BlockSpec((d, d), full),
            pl.BlockSpec((d, d), full),
            pl.BlockSpec((1, d), full),
            pl.BlockSpec((d, 4), full),
            pl.BlockSpec((1, 4), full),
        ],
        out_specs=[pl.BlockSpec((blk, 4), rs)],
        out_shape=[jax.ShapeDtypeStruct((n_pad, 4), jnp.float32)],
    )(agg, agg, inv, h1, w_l, w_r, b, wc2, bc4)


def kernel(x, edge_index, W1_l, b1, W1_r, W2_l, b2, W2_r, Wc, bc):
    n, d = x.shape
    e = edge_index.shape[1]
    ep = e // _NW
    k = 64
    nbc = 16
    nb = -(-ep // k)
    nb = -(-nb // nbc) * nbc
    pad = nb * k - ep
    assert ep % _LANES == 0 and (ep * 4) % 8 == 0

    n_pad = -(-(n + 1) // 128) * 128
    row = edge_index[0].astype(jnp.int32)
    col = edge_index[1].astype(jnp.int32)
    row_p = jnp.pad(row.reshape(_NW, ep), ((0, 0), (0, pad))).reshape(
        _NW, nb, k)
    dummy = (n + jnp.arange(_NW, dtype=jnp.int32) % (n_pad - n))[:, None]
    col_p = jnp.concatenate(
        [col.reshape(_NW, ep),
         jnp.broadcast_to(dummy, (_NW, pad))], axis=1).reshape(_NW, nb, k)
    xp = jnp.pad(x, ((0, n_pad - n), (0, 0)))
    zrow = jnp.zeros((n_pad // _NS, d), jnp.float32)
    zdeg = jnp.zeros((1, n_pad), jnp.float32)

    agg1, degp = _make_sc_agg(n_pad, d, nb, nbc, k, True)(
        xp, row_p, col_p, zrow, zdeg)
    deg_t = degp.reshape(_NW, n_pad).T
    h1, inv = _tc_layer1(agg1, deg_t, xp, W1_l, W1_r, b1.reshape(1, d))

    (agg2,) = _make_sc_agg(n_pad, d, nb, nbc, k, False)(
        h1, row_p, col_p, zrow)
    wc2 = jnp.concatenate([Wc[:d], Wc[d:]], axis=1)
    bc4 = jnp.concatenate([bc, jnp.zeros((2,), jnp.float32)]).reshape(1, 4)
    (pq,) = _tc_layer2(agg2, inv, h1, W2_l, W2_r, b2.reshape(1, d), wc2, bc4)

    (out,) = _make_sc_edge(n_pad, e, ep)(pq.reshape(4 * n_pad), row, col)
    return out.reshape(e, 2)


# speedup vs baseline: 4.6259x; 1.0172x over previous
import functools

import jax
import jax.numpy as jnp
from jax import lax
from jax.experimental import pallas as pl
from jax.experimental.pallas import tpu as pltpu
from jax.experimental.pallas import tpu_sc as plsc

_NC, _NS = 2, 16
_NW = _NC * _NS
_LANES = 16


def _sc_mesh():
    return plsc.VectorSubcoreMesh(
        core_axis_name="c", subcore_axis_name="s",
        num_cores=_NC, num_subcores=_NS)


def _make_sc_agg(n_pad, d, nb, nbc, k, with_deg):
    rpt = n_pad // _NS

    out_type = [jax.ShapeDtypeStruct((_NC, n_pad, d), jnp.float32)]
    scratch = [
        pltpu.VMEM((nbc, k), jnp.int32),
        pltpu.VMEM((nbc, k), jnp.int32),
        pltpu.VMEM((k, d), jnp.float32),
        pltpu.VMEM((k, d), jnp.float32),
        pltpu.SemaphoreType.DMA,
        pltpu.SemaphoreType.DMA,
        pltpu.VMEM_SHARED((n_pad, d), jnp.float32),
    ]
    if with_deg:
        out_type.append(jax.ShapeDtypeStruct((_NW, 1, n_pad), jnp.float32))
        scratch.append(pltpu.VMEM((n_pad,), jnp.float32))

    def body(*refs):
        deg_v = None
        if with_deg:
            (src_hbm, row_hbm, col_hbm, zrow_hbm, zdeg_hbm,
             agg_hbm, deg_hbm, row_v, col_v, g0, g1, sg0, sg1,
             acc_sh, deg_v) = refs
        else:
            (src_hbm, row_hbm, col_hbm, zrow_hbm,
             agg_hbm, row_v, col_v, g0, g1, sg0, sg1, acc_sh) = refs
        gbufs, sems = (g0, g1), (sg0, sg1)
        c = lax.axis_index("c")
        s = lax.axis_index("s")
        wid = c * _NS + s

        pltpu.sync_copy(zrow_hbm, acc_sh.at[pl.ds(s * rpt, rpt)])
        if with_deg:
            pltpu.sync_copy(zdeg_hbm.at[0], deg_v)
        plsc.subcore_barrier()

        ones16 = jnp.ones((_LANES,), jnp.float32)
        nbuf = 2

        def chunk(ci, carry):
            pltpu.sync_copy(row_hbm.at[wid, pl.ds(ci * nbc, nbc)], row_v)
            pltpu.sync_copy(col_hbm.at[wid, pl.ds(ci * nbc, nbc)], col_v)
            cps = [
                pltpu.async_copy(src_hbm.at[row_v.at[j]], gbufs[j], sems[j])
                for j in range(nbuf)
            ]
            for j in range(nbc):
                jj = j % nbuf
                if with_deg:
                    for t in range(k // _LANES):
                        cj = col_v[j, pl.ds(t * _LANES, _LANES)]
                        plsc.addupdate_scatter(deg_v, [cj], ones16)
                cps[jj].wait()
                pltpu.sync_copy(gbufs[jj], acc_sh.at[col_v.at[j]], add=True)
                if j + nbuf < nbc:
                    cps[jj] = pltpu.async_copy(
                        src_hbm.at[row_v.at[j + nbuf]], gbufs[jj], sems[jj])
            return carry

        lax.fori_loop(0, nb // nbc, chunk, 0)

        plsc.subcore_barrier()
        pltpu.sync_copy(acc_sh.at[pl.ds(s * rpt, rpt)],
                        agg_hbm.at[c, pl.ds(s * rpt, rpt)])
        if with_deg:
            pltpu.sync_copy(deg_v, deg_hbm.at[wid, 0])

    return pl.kernel(body, out_type=out_type, mesh=_sc_mesh(),
                     scratch_types=scratch,
                     compiler_params=pltpu.CompilerParams(
                         needs_layout_passes=False))


def _make_sc_edge(n_pad, e, ep):
    ni = ep // _LANES
    out_type = [jax.ShapeDtypeStruct((2 * e,), jnp.float32)]
    scratch = [
        pltpu.VMEM((4 * n_pad,), jnp.float32),
        pltpu.VMEM((ep,), jnp.int32),
        pltpu.VMEM((ep,), jnp.int32),
        pltpu.VMEM((2 * ep,), jnp.float32),
    ]

    def body(pq_hbm, row_hbm, col_hbm, out_hbm, pq_v, row_v, col_v, out_v):
        c = lax.axis_index("c")
        s = lax.axis_index("s")
        wid = c * _NS + s
        base = wid * ep

        pltpu.sync_copy(pq_hbm, pq_v)
        pltpu.sync_copy(row_hbm.at[pl.ds(base, ep)], row_v)
        pltpu.sync_copy(col_hbm.at[pl.ds(base, ep)], col_v)

        iota2 = lax.iota(jnp.int32, _LANES) * 2
        c1 = jnp.ones((_LANES,), jnp.int32)

        def step(i, carry):
            r = row_v[pl.ds(i * _LANES, _LANES)] * 4
            cc = col_v[pl.ds(i * _LANES, _LANES)] * 4 + 2
            o0 = plsc.load_gather(pq_v, [r]) + plsc.load_gather(pq_v, [cc])
            o1 = (plsc.load_gather(pq_v, [r + c1])
                  + plsc.load_gather(pq_v, [cc + c1]))
            pos = i * (2 * _LANES) + iota2
            plsc.store_scatter(out_v, [pos], o0)
            plsc.store_scatter(out_v, [pos + c1], o1)
            return carry

        lax.fori_loop(0, ni, step, 0)
        pltpu.sync_copy(out_v, out_hbm.at[pl.ds(2 * base, 2 * ep)])

    return pl.kernel(body, out_type=out_type, mesh=_sc_mesh(),
                     scratch_types=scratch,
                     compiler_params=pltpu.CompilerParams(
                         needs_layout_passes=False))


def _tc_blk(n_pad):
    best = 8
    for cand in range(8, 2049, 8):
        if n_pad % cand == 0:
            best = cand
    return best


def _tc_layer1(agg, deg_t, x, w_l, w_r, b):
    n_pad, d = x.shape
    blk = _tc_blk(n_pad)
    rs = lambda i: (i, 0)
    full = lambda i: (0, 0)

    def body(a0_ref, a1_ref, degt_ref, x_ref, wl_ref, wr_ref, b_ref,
             h_ref, inv_ref):
        deg = jnp.sum(degt_ref[...], axis=1, keepdims=True)
        inv = 1.0 / jnp.maximum(deg, 1.0)
        mean = (a0_ref[0] + a1_ref[0]) * inv
        h = (jnp.dot(mean, wl_ref[...], preferred_element_type=jnp.float32)
             + jnp.dot(x_ref[...], wr_ref[...], preferred_element_type=jnp.float32)
             + b_ref[...])
        h_ref[...] = jnp.maximum(h, 0.0)
        inv_ref[...] = inv

    return pl.pallas_call(
        body,
        grid=(n_pad // blk,),
        in_specs=[
            pl.BlockSpec((1, blk, d), lambda i: (0, i, 0)),
            pl.BlockSpec((1, blk, d), lambda i: (1, i, 0)),
            pl.BlockSpec((blk, _NW), rs),
            pl.BlockSpec((blk, d), rs),
            pl.BlockSpec((d, d), full),
            pl.BlockSpec((d, d), full),
            pl.BlockSpec((1, d), full),
        ],
        out_specs=[pl.BlockSpec((blk, d), rs), pl.BlockSpec((blk, 1), rs)],
        out_shape=[jax.ShapeDtypeStruct((n_pad, d), jnp.float32),
                   jax.ShapeDtypeStruct((n_pad, 1), jnp.float32)],
    )(agg, agg, deg_t, x, w_l, w_r, b)


def _tc_layer2(agg, inv, h1, w_l, w_r, b, wc2, bc4):
    n_pad, d = h1.shape
    blk = _tc_blk(n_pad)
    rs = lambda i: (i, 0)
    full = lambda i: (0, 0)

    def body(a0_ref, a1_ref, inv_ref, h1_ref, wl_ref, wr_ref, b_ref,
             wc_ref, bc_ref, pq_ref):
        mean = (a0_ref[0] + a1_ref[0]) * inv_ref[...]
        h2 = (jnp.dot(mean, wl_ref[...], preferred_element_type=jnp.float32)
              + jnp.dot(h1_ref[...], wr_ref[...], preferred_element_type=jnp.float32)
              + b_ref[...])
        pq_ref[...] = (jnp.dot(h2, wc_ref[...],
                               preferred_element_type=jnp.float32)
                       + bc_ref[...])

    return pl.pallas_call(
        body,
        grid=(n_pad // blk,),
        in_specs=[
            pl.BlockSpec((1, blk, d), lambda i: (0, i, 0)),
            pl.BlockSpec((1, blk, d), lambda i: (1, i, 0)),
            pl.BlockSpec((blk, 1), rs),
            pl.BlockSpec((blk, d), rs),
            pl.BlockSpec((d, d), full),
            pl.BlockSpec((d, d), full),
            pl.BlockSpec((1, d), full),
            pl.BlockSpec((d, 4), full),
            pl.BlockSpec((1, 4), full),
        ],
        out_specs=[pl.BlockSpec((blk, 4), rs)],
        out_shape=[jax.ShapeDtypeStruct((n_pad, 4), jnp.float32)],
    )(agg, agg, inv, h1, w_l, w_r, b, wc2, bc4)


def kernel(x, edge_index, W1_l, b1, W1_r, W2_l, b2, W2_r, Wc, bc):
    n, d = x.shape
    e = edge_index.shape[1]
    ep = e // _NW
    k = 64
    nbc = 16
    nb = -(-ep // k)
    nb = -(-nb // nbc) * nbc
    pad = nb * k - ep
    assert ep % _LANES == 0 and (ep * 4) % 8 == 0

    n_pad = -(-(n + 1) // 128) * 128
    k2 = 128
    nbc2 = 8
    nb2 = -(-ep // k2)
    nb2 = -(-nb2 // nbc2) * nbc2
    pad2 = nb2 * k2 - ep

    row = edge_index[0].astype(jnp.int32)
    col = edge_index[1].astype(jnp.int32)
    row_p = jnp.pad(row.reshape(_NW, ep), ((0, 0), (0, pad))).reshape(
        _NW, nb, k)
    dummy = (n + jnp.arange(_NW, dtype=jnp.int32) % (n_pad - n))[:, None]
    col_p = jnp.concatenate(
        [col.reshape(_NW, ep),
         jnp.broadcast_to(dummy, (_NW, pad))], axis=1).reshape(_NW, nb, k)
    row_p2 = jnp.pad(row.reshape(_NW, ep), ((0, 0), (0, pad2))).reshape(
        _NW, nb2, k2)
    col_p2 = jnp.concatenate(
        [col.reshape(_NW, ep),
         jnp.broadcast_to(dummy, (_NW, pad2))], axis=1).reshape(_NW, nb2, k2)
    zrow = jnp.zeros((n_pad // _NS, d), jnp.float32)
    zdeg = jnp.zeros((1, n_pad), jnp.float32)

    agg1, degp = _make_sc_agg(n_pad, d, nb, nbc, k, True)(
        x, row_p, col_p, zrow, zdeg)
    deg_t = degp.reshape(_NW, n_pad).T
    h1, inv = _tc_layer1(agg1, deg_t, x, W1_l, W1_r, b1.reshape(1, d))

    (agg2,) = _make_sc_agg(n_pad, d, nb2, nbc2, k2, False)(
        h1, row_p2, col_p2, zrow)
    wc2 = jnp.concatenate([Wc[:d], Wc[d:]], axis=1)
    bc4 = jnp.concatenate([bc, jnp.zeros((2,), jnp.float32)]).reshape(1, 4)
    (pq,) = _tc_layer2(agg2, inv, h1, W2_l, W2_r, b2.reshape(1, d), wc2, bc4)

    (out,) = _make_sc_edge(n, e, ep)(pq.reshape(4 * n), row, col)
    return out.reshape(e, 2)


# speedup vs baseline: 4.6418x; 1.0034x over previous
import functools

import jax
import jax.numpy as jnp
from jax import lax
from jax.experimental import pallas as pl
from jax.experimental.pallas import tpu as pltpu
from jax.experimental.pallas import tpu_sc as plsc

_NC, _NS = 2, 16
_NW = _NC * _NS
_LANES = 16


def _sc_mesh():
    return plsc.VectorSubcoreMesh(
        core_axis_name="c", subcore_axis_name="s",
        num_cores=_NC, num_subcores=_NS)


def _make_sc_agg(n_pad, d, nb, nbc, k, with_deg):
    rpt = n_pad // _NS

    out_type = [jax.ShapeDtypeStruct((_NC, n_pad, d), jnp.float32)]
    scratch = [
        pltpu.VMEM((nbc, k), jnp.int32),
        pltpu.VMEM((nbc, k), jnp.int32),
        pltpu.VMEM((k, d), jnp.float32),
        pltpu.VMEM((k, d), jnp.float32),
        pltpu.SemaphoreType.DMA,
        pltpu.SemaphoreType.DMA,
        pltpu.VMEM_SHARED((n_pad, d), jnp.float32),
    ]
    if with_deg:
        out_type.append(jax.ShapeDtypeStruct((_NW, 1, n_pad), jnp.float32))
        scratch.append(pltpu.VMEM((n_pad,), jnp.float32))

    def body(*refs):
        deg_v = None
        if with_deg:
            (src_hbm, row_hbm, col_hbm,
             agg_hbm, deg_hbm, row_v, col_v, g0, g1, sg0, sg1,
             acc_sh, deg_v) = refs
        else:
            (src_hbm, row_hbm, col_hbm,
             agg_hbm, row_v, col_v, g0, g1, sg0, sg1, acc_sh) = refs
        gbufs, sems = (g0, g1), (sg0, sg1)
        c = lax.axis_index("c")
        s = lax.axis_index("s")
        wid = c * _NS + s

        zeros16 = jnp.zeros((_LANES,), jnp.float32)
        npl = d // _LANES

        def zstep(i, carry):
            g0[i // npl, pl.ds((i % npl) * _LANES, _LANES)] = zeros16
            return carry

        lax.fori_loop(0, k * npl, zstep, 0)
        off = 0
        while off < rpt:
            rows = min(k, rpt - off)
            pltpu.sync_copy(g0.at[pl.ds(0, rows)],
                            acc_sh.at[pl.ds(s * rpt + off, rows)])
            off += rows
        if with_deg:

            def zdeg_step(i, carry):
                deg_v[pl.ds(i * _LANES, _LANES)] = zeros16
                return carry

            lax.fori_loop(0, n_pad // _LANES, zdeg_step, 0)
        plsc.subcore_barrier()

        ones16 = jnp.ones((_LANES,), jnp.float32)
        nbuf = 2

        def chunk(ci, carry):
            pltpu.sync_copy(row_hbm.at[wid, pl.ds(ci * nbc, nbc)], row_v)
            pltpu.sync_copy(col_hbm.at[wid, pl.ds(ci * nbc, nbc)], col_v)
            cps = [
                pltpu.async_copy(src_hbm.at[row_v.at[j]], gbufs[j], sems[j])
                for j in range(nbuf)
            ]
            for j in range(nbc):
                jj = j % nbuf
                if with_deg:
                    for t in range(k // _LANES):
                        cj = col_v[j, pl.ds(t * _LANES, _LANES)]
                        plsc.addupdate_scatter(deg_v, [cj], ones16)
                cps[jj].wait()
                pltpu.sync_copy(gbufs[jj], acc_sh.at[col_v.at[j]], add=True)
                if j + nbuf < nbc:
                    cps[jj] = pltpu.async_copy(
                        src_hbm.at[row_v.at[j + nbuf]], gbufs[jj], sems[jj])
            return carry

        lax.fori_loop(0, nb // nbc, chunk, 0)

        plsc.subcore_barrier()
        pltpu.sync_copy(acc_sh.at[pl.ds(s * rpt, rpt)],
                        agg_hbm.at[c, pl.ds(s * rpt, rpt)])
        if with_deg:
            pltpu.sync_copy(deg_v, deg_hbm.at[wid, 0])

    return pl.kernel(body, out_type=out_type, mesh=_sc_mesh(),
                     scratch_types=scratch,
                     compiler_params=pltpu.CompilerParams(
                         needs_layout_passes=False))


def _make_sc_edge(n_pad, e, ep):
    ni = ep // _LANES
    out_type = [jax.ShapeDtypeStruct((2 * e,), jnp.float32)]
    scratch = [
        pltpu.VMEM((4 * n_pad,), jnp.float32),
        pltpu.VMEM((ep,), jnp.int32),
        pltpu.VMEM((ep,), jnp.int32),
        pltpu.VMEM((2 * ep,), jnp.float32),
    ]

    def body(pq_hbm, row_hbm, col_hbm, out_hbm, pq_v, row_v, col_v, out_v):
        c = lax.axis_index("c")
        s = lax.axis_index("s")
        wid = c * _NS + s
        base = wid * ep

        pltpu.sync_copy(pq_hbm, pq_v)
        pltpu.sync_copy(row_hbm.at[pl.ds(base, ep)], row_v)
        pltpu.sync_copy(col_hbm.at[pl.ds(base, ep)], col_v)

        iota2 = lax.iota(jnp.int32, _LANES) * 2
        c1 = jnp.ones((_LANES,), jnp.int32)

        def step(i, carry):
            r = row_v[pl.ds(i * _LANES, _LANES)] * 4
            cc = col_v[pl.ds(i * _LANES, _LANES)] * 4 + 2
            o0 = plsc.load_gather(pq_v, [r]) + plsc.load_gather(pq_v, [cc])
            o1 = (plsc.load_gather(pq_v, [r + c1])
                  + plsc.load_gather(pq_v, [cc + c1]))
            pos = i * (2 * _LANES) + iota2
            plsc.store_scatter(out_v, [pos], o0)
            plsc.store_scatter(out_v, [pos + c1], o1)
            return carry

        lax.fori_loop(0, ni, step, 0)
        pltpu.sync_copy(out_v, out_hbm.at[pl.ds(2 * base, 2 * ep)])

    return pl.kernel(body, out_type=out_type, mesh=_sc_mesh(),
                     scratch_types=scratch,
                     compiler_params=pltpu.CompilerParams(
                         needs_layout_passes=False))


def _tc_blk(n_pad):
    best = 8
    for cand in range(8, 2049, 8):
        if n_pad % cand == 0:
            best = cand
    return best


def _tc_layer1(agg, deg_t, x, w_l, w_r, b):
    n_pad, d = x.shape
    blk = _tc_blk(n_pad)
    rs = lambda i: (i, 0)
    full = lambda i: (0, 0)

    def body(a0_ref, a1_ref, degt_ref, x_ref, wl_ref, wr_ref, b_ref,
             h_ref, inv_ref):
        deg = jnp.sum(degt_ref[...], axis=1, keepdims=True)
        inv = 1.0 / jnp.maximum(deg, 1.0)
        mean = (a0_ref[0] + a1_ref[0]) * inv
        h = (jnp.dot(mean, wl_ref[...], preferred_element_type=jnp.float32)
             + jnp.dot(x_ref[...], wr_ref[...], preferred_element_type=jnp.float32)
             + b_ref[...])
        h_ref[...] = jnp.maximum(h, 0.0)
        inv_ref[...] = inv

    return pl.pallas_call(
        body,
        grid=(n_pad // blk,),
        in_specs=[
            pl.BlockSpec((1, blk, d), lambda i: (0, i, 0)),
            pl.BlockSpec((1, blk, d), lambda i: (1, i, 0)),
            pl.BlockSpec((blk, _NW), rs),
            pl.BlockSpec((blk, d), rs),
            pl.BlockSpec((d, d), full),
            pl.BlockSpec((d, d), full),
            pl.BlockSpec((1, d), full),
        ],
        out_specs=[pl.BlockSpec((blk, d), rs), pl.BlockSpec((blk, 1), rs)],
        out_shape=[jax.ShapeDtypeStruct((n_pad, d), jnp.float32),
                   jax.ShapeDtypeStruct((n_pad, 1), jnp.float32)],
    )(agg, agg, deg_t, x, w_l, w_r, b)


def _tc_layer2(agg, inv, h1, w_l, w_r, b, wc2, bc4):
    n_pad, d = h1.shape
    blk = _tc_blk(n_pad)
    rs = lambda i: (i, 0)
    full = lambda i: (0, 0)

    def body(a0_ref, a1_ref, inv_ref, h1_ref, wl_ref, wr_ref, b_ref,
             wc_ref, bc_ref, pq_ref):
        mean = (a0_ref[0] + a1_ref[0]) * inv_ref[...]
        h2 = (jnp.dot(mean, wl_ref[...], preferred_element_type=jnp.float32)
              + jnp.dot(h1_ref[...], wr_ref[...], preferred_element_type=jnp.float32)
              + b_ref[...])
        pq_ref[...] = (jnp.dot(h2, wc_ref[...],
                               preferred_element_type=jnp.float32)
                       + bc_ref[...])

    return pl.pallas_call(
        body,
        grid=(n_pad // blk,),
        in_specs=[
            pl.BlockSpec((1, blk, d), lambda i: (0, i, 0)),
            pl.BlockSpec((1, blk, d), lambda i: (1, i, 0)),
            pl.BlockSpec((blk, 1), rs),
            pl.BlockSpec((blk, d), rs),
            pl.BlockSpec((d, d), full),
            pl.BlockSpec((d, d), full),
            pl.BlockSpec((1, d), full),
            pl.BlockSpec((d, 4), full),
            pl.BlockSpec((1, 4), full),
        ],
        out_specs=[pl.BlockSpec((blk, 4), rs)],
        out_shape=[jax.ShapeDtypeStruct((n_pad, 4), jnp.float32)],
    )(agg, agg, inv, h1, w_l, w_r, b, wc2, bc4)


def kernel(x, edge_index, W1_l, b1, W1_r, W2_l, b2, W2_r, Wc, bc):
    n, d = x.shape
    e = edge_index.shape[1]
    ep = e // _NW
    k = 64
    nbc = 16
    nb = -(-ep // k)
    nb = -(-nb // nbc) * nbc
    pad = nb * k - ep
    assert ep % _LANES == 0 and (ep * 4) % 8 == 0

    n_pad = -(-(n + 1) // 128) * 128
    k2 = 128
    nbc2 = 8
    nb2 = -(-ep // k2)
    nb2 = -(-nb2 // nbc2) * nbc2
    pad2 = nb2 * k2 - ep

    row = edge_index[0].astype(jnp.int32)
    col = edge_index[1].astype(jnp.int32)
    row_p = jnp.pad(row.reshape(_NW, ep), ((0, 0), (0, pad))).reshape(
        _NW, nb, k)
    dummy = (n + jnp.arange(_NW, dtype=jnp.int32) % (n_pad - n))[:, None]
    col_p = jnp.concatenate(
        [col.reshape(_NW, ep),
         jnp.broadcast_to(dummy, (_NW, pad))], axis=1).reshape(_NW, nb, k)
    row_p2 = jnp.pad(row.reshape(_NW, ep), ((0, 0), (0, pad2))).reshape(
        _NW, nb2, k2)
    col_p2 = jnp.concatenate(
        [col.reshape(_NW, ep),
         jnp.broadcast_to(dummy, (_NW, pad2))], axis=1).reshape(_NW, nb2, k2)
    agg1, degp = _make_sc_agg(n_pad, d, nb, nbc, k, True)(
        x, row_p, col_p)
    deg_t = degp.reshape(_NW, n_pad).T
    h1, inv = _tc_layer1(agg1, deg_t, x, W1_l, W1_r, b1.reshape(1, d))

    (agg2,) = _make_sc_agg(n_pad, d, nb2, nbc2, k2, False)(
        h1, row_p2, col_p2)
    wc2 = jnp.concatenate([Wc[:d], Wc[d:]], axis=1)
    bc4 = jnp.concatenate([bc, jnp.zeros((2,), jnp.float32)]).reshape(1, 4)
    (pq,) = _tc_layer2(agg2, inv, h1, W2_l, W2_r, b2.reshape(1, d), wc2, bc4)

    (out,) = _make_sc_edge(n, e, ep)(pq.reshape(4 * n), row, col)
    return out.reshape(e, 2)
